# baseline (speedup 1.0000x reference)
import functools

import jax
import jax.numpy as jnp
from jax import lax
from jax.experimental import pallas as pl
from jax.experimental.pallas import tpu as pltpu
from jax.experimental.pallas import tpu_sc as plsc

N = 50000
E = 800000
WIDTH = 64
NQ = 2
QW = WIDTH // NQ
DEPTH = 4

LANES = 128
N_PAD = 50176
EROWS = 6400
E_PAD = EROWS * LANES
RPT = N_PAD // 16
ERPT = EROWS // 16
EPT = ERPT * LANES
SUP = 1024
SPT = EPT // SUP
SROWS = SUP // LANES
GC = 256
GPS = SUP // GC


EDPT = EROWS // 32
DW = 8


def _sc_degree_body(dst2d, ones2d, degp, dstv, onesv, d_sh, sem):
    cid = lax.axis_index("c")
    sid = lax.axis_index("s")
    pltpu.sync_copy(ones2d.at[pl.ds(sid * RPT, RPT)],
                    d_sh.at[pl.ds(sid * RPT, RPT)])
    pltpu.sync_copy(ones2d.at[pl.ds(0, LANES)], onesv)
    pltpu.sync_copy(
        dst2d.at[pl.ds(cid * (EROWS // 2) + sid * EDPT, EDPT)], dstv)
    plsc.subcore_barrier()

    def step(j, carry):
        pltpu.sync_copy(onesv, d_sh.at[dstv.at[j]], add=True)
        return carry

    lax.fori_loop(0, EDPT, step, 0)
    plsc.subcore_barrier()

    def writeout(c):
        pltpu.sync_copy(d_sh.at[pl.ds(sid * RPT, RPT)],
                        degp.at[c, pl.ds(sid * RPT, RPT)])

    pl.when(cid == 0)(lambda: writeout(0))
    pl.when(cid == 1)(lambda: writeout(1))


def _sc_scatter_body(*refs):
    y0, y1, src_flat, dst2d, z0, z1 = refs[:6]
    srcb = refs[6:8]
    dstb = refs[8:10]
    rows = refs[10:12]
    z_sh = refs[12]
    isem = refs[13:15]
    gsem = refs[15:17]
    ssem = refs[17:19]
    cid = lax.axis_index("c")
    sid = lax.axis_index("s")
    src_base = sid * EPT
    drow_base = sid * ERPT

    def idxload(s, h):
        pltpu.async_copy(src_flat.at[pl.ds(src_base + s * SUP, SUP)],
                         srcb[h], isem[h])
        pltpu.async_copy(dst2d.at[pl.ds(drow_base + s * SROWS, SROWS)],
                         dstb[h], isem[h])

    def idxwait(s, h):
        pltpu.make_async_copy(src_flat.at[pl.ds(src_base + s * SUP, SUP)],
                              srcb[h], isem[h]).wait()
        pltpu.make_async_copy(dst2d.at[pl.ds(drow_base + s * SROWS, SROWS)],
                              dstb[h], isem[h]).wait()

    def half(y_hbm, z_hbm):
        idxload(0, 0)
        idxload(1, 1)
        pltpu.sync_copy(y_hbm.at[pl.ds(sid * RPT, RPT)],
                        z_sh.at[pl.ds(sid * RPT, RPT)])

        def gather(h, q, rb):
            pltpu.async_copy(
                y_hbm.at[srcb[h].at[pl.ds(q * GC, GC)]], rows[rb], gsem[rb])

        def gwait(h, q, rb):
            pltpu.make_async_copy(
                y_hbm.at[srcb[h].at[pl.ds(q * GC, GC)]],
                rows[rb], gsem[rb]).wait()

        def scatter_issue(h, q, rb):
            for i in range(GC // LANES):
                pltpu.async_copy(
                    rows[rb].at[pl.ds(i * LANES, LANES)],
                    z_sh.at[dstb[h].at[q * (GC // LANES) + i]],
                    ssem[rb], add=True)

        def scatter_wait(h, rb):
            for i in range(GC // LANES):
                pltpu.make_async_copy(
                    rows[rb].at[pl.ds(i * LANES, LANES)],
                    z_sh.at[dstb[h].at[i]], ssem[rb]).wait()

        idxwait(0, 0)
        plsc.subcore_barrier()
        gather(0, 0, 0)

        def process(s, h, maybe_first):
            for q in range(GPS):
                rb = q % 2
                gwait(h, q, rb)
                scatter_issue(h, q, rb)
                if q == 0 and maybe_first:
                    @pl.when(s > 0)
                    def _():
                        scatter_wait(h, 1 - rb)
                else:
                    scatter_wait(h, 1 - rb)
                if q < GPS - 1:
                    gather(h, q + 1, 1 - rb)
                else:
                    @pl.when(s + 1 < SPT)
                    def _(h=h, rb=rb):
                        idxwait(s + 1, 1 - h)
                        gather(1 - h, 0, 1 - rb)

            @pl.when(s + 2 < SPT)
            def _(s=s, h=h):
                idxload(s + 2, h)

        def superpair(p, carry):
            process(2 * p, 0, True)
            process(2 * p + 1, 1, False)
            return carry

        lax.fori_loop(0, SPT // 2, superpair, 0)
        scatter_wait(1, 1)
        plsc.subcore_barrier()
        pltpu.sync_copy(z_sh.at[pl.ds(sid * RPT, RPT)],
                        z_hbm.at[pl.ds(sid * RPT, RPT)])

    pl.when(cid == 0)(lambda: half(y0, z0))
    pl.when(cid == 1)(lambda: half(y1, z1))


@functools.cache
def _sc_kernels():
    mesh = plsc.VectorSubcoreMesh(core_axis_name="c", subcore_axis_name="s")
    params = pltpu.CompilerParams(use_tc_tiling_on_sc=False)
    degree = pl.kernel(
        _sc_degree_body,
        out_type=jax.ShapeDtypeStruct((2, N_PAD, DW), jnp.float32),
        mesh=mesh,
        scratch_types=[
            pltpu.VMEM((EDPT, LANES), jnp.int32),
            pltpu.VMEM((LANES, DW), jnp.float32),
            pltpu.VMEM_SHARED((N_PAD, DW), jnp.float32),
            pltpu.SemaphoreType.DMA,
        ],
        compiler_params=params,
    )
    qshape = jax.ShapeDtypeStruct((N_PAD, QW), jnp.float32)
    scatter = pl.kernel(
        _sc_scatter_body,
        out_type=[qshape] * NQ,
        mesh=mesh,
        scratch_types=(
            [pltpu.VMEM((SUP,), jnp.int32)] * 2
            + [pltpu.VMEM((SROWS, LANES), jnp.int32)] * 2
            + [pltpu.VMEM((GC, QW), jnp.float32)] * 2
            + [pltpu.VMEM_SHARED((N_PAD, QW), jnp.float32)]
            + [pltpu.SemaphoreType.DMA] * 6
        ),
        compiler_params=params,
    )
    return degree, scatter


BN = 1024
GRID = N_PAD // BN


def _split(y, outs):
    for q, ref in enumerate(outs):
        ref[...] = y[:, q * QW:(q + 1) * QW]


def _tc_pre_body(x, degp, fc1_W, fc1_b, conv_W, *outs):
    di = lax.rsqrt(degp[0, :, 0:1] + degp[1, :, 0:1] - 1.0)
    h = jnp.dot(x[...], fc1_W[...], preferred_element_type=jnp.float32)
    h = h + fc1_b[...]
    y = jnp.dot(h, conv_W[...], preferred_element_type=jnp.float32) * di
    _split(y, outs[:NQ])
    outs[NQ][...] = di


_qspec = pl.BlockSpec((BN, QW), lambda i: (i, 0))
_qshape = jax.ShapeDtypeStruct((N_PAD, QW), jnp.float32)
_dspec = pl.BlockSpec((BN, 1), lambda i: (i, 0))

_tc_pre = pl.pallas_call(
    _tc_pre_body,
    grid=(GRID,),
    in_specs=[
        pl.BlockSpec((BN, 3), lambda i: (i, 0)),
        pl.BlockSpec((2, BN, DW), lambda i: (0, i, 0)),
        pl.BlockSpec((3, WIDTH), lambda i: (0, 0)),
        pl.BlockSpec((1, WIDTH), lambda i: (0, 0)),
        pl.BlockSpec((WIDTH, WIDTH), lambda i: (0, 0)),
    ],
    out_specs=[_qspec] * NQ + [_dspec],
    out_shape=[_qshape] * NQ + [jax.ShapeDtypeStruct((N_PAD, 1), jnp.float32)],
)


def _tc_mid_body(*refs):
    zs = refs[:NQ]
    dinv, conv_W, conv_b = refs[NQ:NQ + 3]
    ys = refs[NQ + 3:]
    di = dinv[...]
    z = jnp.concatenate([zq[...] for zq in zs], axis=1)
    h = jnp.maximum(z * di + conv_b[...], 0.0)
    y = jnp.dot(h, conv_W[...], preferred_element_type=jnp.float32) * di
    _split(y, ys)


_tc_mid = pl.pallas_call(
    _tc_mid_body,
    grid=(GRID,),
    in_specs=[_qspec] * NQ + [
        _dspec,
        pl.BlockSpec((WIDTH, WIDTH), lambda i: (0, 0)),
        pl.BlockSpec((1, WIDTH), lambda i: (0, 0)),
    ],
    out_specs=[_qspec] * NQ,
    out_shape=[_qshape] * NQ,
)


def _tc_post_body(*refs):
    zs = refs[:NQ]
    dinv, conv_b, fc2_W, fc2_b, out = refs[NQ:]
    di = dinv[...]
    z = jnp.concatenate([zq[...] for zq in zs], axis=1)
    h = jnp.maximum(z * di + conv_b[...], 0.0)
    out[...] = jnp.dot(h, fc2_W[...], preferred_element_type=jnp.float32) + fc2_b[...]


_tc_post = pl.pallas_call(
    _tc_post_body,
    grid=(GRID,),
    in_specs=[_qspec] * NQ + [
        _dspec,
        pl.BlockSpec((1, WIDTH), lambda i: (0, 0)),
        pl.BlockSpec((WIDTH, 1), lambda i: (0, 0)),
        pl.BlockSpec((1, 1), lambda i: (0, 0)),
    ],
    out_specs=pl.BlockSpec((BN, 1), lambda i: (i, 0)),
    out_shape=jax.ShapeDtypeStruct((N_PAD, 1), jnp.float32),
)


def kernel(x, edge_index, fc1_W, fc1_b, conv_W, conv_b, fc2_W, fc2_b):
    src = jnp.concatenate(
        [edge_index[0], jnp.zeros((E_PAD - E,), jnp.int32)])
    dst = jnp.concatenate(
        [edge_index[1], jnp.full((E_PAD - E,), N, jnp.int32)]).reshape(EROWS, LANES)
    x_pad = jnp.concatenate([x, jnp.zeros((N_PAD - N, 3), x.dtype)], axis=0)

    sc_degree, sc_scatter = _sc_kernels()
    fc1_b2 = fc1_b.reshape(1, WIDTH)
    conv_b2 = conv_b.reshape(1, WIDTH)

    degp = sc_degree(dst, jnp.ones((N_PAD, DW), jnp.float32))
    outs = _tc_pre(x_pad, degp, fc1_W, fc1_b2, conv_W)
    ys, dinv = outs[:NQ], outs[NQ]
    for layer in range(DEPTH):
        zs = sc_scatter(*ys, src, dst)
        if layer < DEPTH - 1:
            ys = _tc_mid(*zs, dinv, conv_W, conv_b2)
    out = _tc_post(*zs, dinv, conv_b2, fc2_W, fc2_b.reshape(1, 1))
    return out[:N]


# speedup vs baseline: 1.0329x; 1.0329x over previous
import functools

import jax
import jax.numpy as jnp
from jax import lax
from jax.experimental import pallas as pl
from jax.experimental.pallas import tpu as pltpu
from jax.experimental.pallas import tpu_sc as plsc

N = 50000
E = 800000
WIDTH = 64
NQ = 2
QW = WIDTH // NQ
DEPTH = 4

LANES = 128
N_PAD = 50176
EROWS = 6400
E_PAD = EROWS * LANES
RPT = N_PAD // 16
ERPT = EROWS // 16
EPT = ERPT * LANES
SUP = 1024
SPT = EPT // SUP
SROWS = SUP // LANES
GC = 256
GPS = SUP // GC


EDPT = EROWS // 32
DW = 8


def _sc_degree_body(dst2d, ones2d, degp, dstv, onesv, d_sh, sem):
    cid = lax.axis_index("c")
    sid = lax.axis_index("s")
    pltpu.sync_copy(ones2d.at[pl.ds(sid * RPT, RPT)],
                    d_sh.at[pl.ds(sid * RPT, RPT)])
    pltpu.sync_copy(ones2d.at[pl.ds(0, LANES)], onesv)
    pltpu.sync_copy(
        dst2d.at[pl.ds(cid * (EROWS // 2) + sid * EDPT, EDPT)], dstv)
    plsc.subcore_barrier()

    def step(j, carry):
        pltpu.sync_copy(onesv, d_sh.at[dstv.at[j]], add=True)
        return carry

    lax.fori_loop(0, EDPT, step, 0)
    plsc.subcore_barrier()

    def writeout(c):
        pltpu.sync_copy(d_sh.at[pl.ds(sid * RPT, RPT)],
                        degp.at[c, pl.ds(sid * RPT, RPT)])

    pl.when(cid == 0)(lambda: writeout(0))
    pl.when(cid == 1)(lambda: writeout(1))


def _sc_scatter_body(*refs):
    y0, y1, src_flat, dst2d, z0, z1 = refs[:6]
    srcb = refs[6:8]
    dstb = refs[8:10]
    rows = refs[10:12]
    z_sh = refs[12]
    isem = refs[13:15]
    gsem = refs[15:17]
    ssem = refs[17:19]
    cid = lax.axis_index("c")
    sid = lax.axis_index("s")
    src_base = sid * EPT
    drow_base = sid * ERPT

    def idxload(s, h):
        pltpu.async_copy(src_flat.at[pl.ds(src_base + s * SUP, SUP)],
                         srcb[h], isem[h])
        pltpu.async_copy(dst2d.at[pl.ds(drow_base + s * SROWS, SROWS)],
                         dstb[h], isem[h])

    def idxwait(s, h):
        pltpu.make_async_copy(src_flat.at[pl.ds(src_base + s * SUP, SUP)],
                              srcb[h], isem[h]).wait()
        pltpu.make_async_copy(dst2d.at[pl.ds(drow_base + s * SROWS, SROWS)],
                              dstb[h], isem[h]).wait()

    def half(y_hbm, z_hbm):
        idxload(0, 0)
        idxload(1, 1)
        pltpu.sync_copy(y_hbm.at[pl.ds(sid * RPT, RPT)],
                        z_sh.at[pl.ds(sid * RPT, RPT)])

        def gather(h, q, rb):
            pltpu.async_copy(
                y_hbm.at[srcb[h].at[pl.ds(q * GC, GC)]], rows[rb], gsem[rb])

        def gwait(h, q, rb):
            pltpu.make_async_copy(
                y_hbm.at[srcb[h].at[pl.ds(q * GC, GC)]],
                rows[rb], gsem[rb]).wait()

        def scatter_issue(h, q, rb):
            for i in range(GC // LANES):
                pltpu.async_copy(
                    rows[rb].at[pl.ds(i * LANES, LANES)],
                    z_sh.at[dstb[h].at[q * (GC // LANES) + i]],
                    ssem[rb], add=True)

        def scatter_wait(h, rb):
            for i in range(GC // LANES):
                pltpu.make_async_copy(
                    rows[rb].at[pl.ds(i * LANES, LANES)],
                    z_sh.at[dstb[h].at[i]], ssem[rb]).wait()

        idxwait(0, 0)
        plsc.subcore_barrier()
        gather(0, 0, 0)

        def process(s, h, maybe_first):
            for q in range(GPS):
                rb = q % 2
                gwait(h, q, rb)
                scatter_issue(h, q, rb)
                if q == 0 and maybe_first:
                    @pl.when(s > 0)
                    def _():
                        scatter_wait(h, 1 - rb)
                else:
                    scatter_wait(h, 1 - rb)
                if q < GPS - 1:
                    gather(h, q + 1, 1 - rb)
                else:
                    @pl.when(s + 1 < SPT)
                    def _(h=h, rb=rb):
                        idxwait(s + 1, 1 - h)
                        gather(1 - h, 0, 1 - rb)

            @pl.when(s + 2 < SPT)
            def _(s=s, h=h):
                idxload(s + 2, h)

        def superpair(p, carry):
            process(2 * p, 0, True)
            process(2 * p + 1, 1, False)
            return carry

        lax.fori_loop(0, SPT // 2, superpair, 0)
        scatter_wait(1, 1)
        plsc.subcore_barrier()
        pltpu.sync_copy(z_sh.at[pl.ds(sid * RPT, RPT)],
                        z_hbm.at[pl.ds(sid * RPT, RPT)])

    pl.when(cid == 0)(lambda: half(y0, z0))
    pl.when(cid == 1)(lambda: half(y1, z1))


@functools.cache
def _sc_kernels():
    mesh = plsc.VectorSubcoreMesh(core_axis_name="c", subcore_axis_name="s")
    params = pltpu.CompilerParams(use_tc_tiling_on_sc=False)
    degree = pl.kernel(
        _sc_degree_body,
        out_type=jax.ShapeDtypeStruct((2, N_PAD, DW), jnp.float32),
        mesh=mesh,
        scratch_types=[
            pltpu.VMEM((EDPT, LANES), jnp.int32),
            pltpu.VMEM((LANES, DW), jnp.float32),
            pltpu.VMEM_SHARED((N_PAD, DW), jnp.float32),
            pltpu.SemaphoreType.DMA,
        ],
        compiler_params=params,
    )
    qshape = jax.ShapeDtypeStruct((N_PAD, QW), jnp.float32)
    scatter = pl.kernel(
        _sc_scatter_body,
        out_type=[qshape] * NQ,
        mesh=mesh,
        scratch_types=(
            [pltpu.VMEM((SUP,), jnp.int32)] * 2
            + [pltpu.VMEM((SROWS, LANES), jnp.int32)] * 2
            + [pltpu.VMEM((GC, QW), jnp.float32)] * 2
            + [pltpu.VMEM_SHARED((N_PAD, QW), jnp.float32)]
            + [pltpu.SemaphoreType.DMA] * 6
        ),
        compiler_params=params,
    )
    return degree, scatter


BN = 3136
GRID = N_PAD // BN


def _split(y, outs):
    for q, ref in enumerate(outs):
        ref[...] = y[:, q * QW:(q + 1) * QW]


def _tc_pre_body(x, degp, fc1_W, fc1_b, conv_W, *outs):
    di = lax.rsqrt(degp[0, :, 0:1] + degp[1, :, 0:1] - 1.0)
    h = jnp.dot(x[...], fc1_W[...], preferred_element_type=jnp.float32)
    h = h + fc1_b[...]
    y = jnp.dot(h, conv_W[...], preferred_element_type=jnp.float32) * di
    _split(y, outs[:NQ])
    outs[NQ][...] = di


_qspec = pl.BlockSpec((BN, QW), lambda i: (i, 0))
_qshape = jax.ShapeDtypeStruct((N_PAD, QW), jnp.float32)
_dspec = pl.BlockSpec((BN, 1), lambda i: (i, 0))

_tc_pre = pl.pallas_call(
    _tc_pre_body,
    grid=(GRID,),
    in_specs=[
        pl.BlockSpec((BN, 3), lambda i: (i, 0)),
        pl.BlockSpec((2, BN, DW), lambda i: (0, i, 0)),
        pl.BlockSpec((3, WIDTH), lambda i: (0, 0)),
        pl.BlockSpec((1, WIDTH), lambda i: (0, 0)),
        pl.BlockSpec((WIDTH, WIDTH), lambda i: (0, 0)),
    ],
    out_specs=[_qspec] * NQ + [_dspec],
    out_shape=[_qshape] * NQ + [jax.ShapeDtypeStruct((N_PAD, 1), jnp.float32)],
)


def _tc_mid_body(*refs):
    zs = refs[:NQ]
    dinv, conv_W, conv_b = refs[NQ:NQ + 3]
    ys = refs[NQ + 3:]
    di = dinv[...]
    z = jnp.concatenate([zq[...] for zq in zs], axis=1)
    h = jnp.maximum(z * di + conv_b[...], 0.0)
    y = jnp.dot(h, conv_W[...], preferred_element_type=jnp.float32) * di
    _split(y, ys)


_tc_mid = pl.pallas_call(
    _tc_mid_body,
    grid=(GRID,),
    in_specs=[_qspec] * NQ + [
        _dspec,
        pl.BlockSpec((WIDTH, WIDTH), lambda i: (0, 0)),
        pl.BlockSpec((1, WIDTH), lambda i: (0, 0)),
    ],
    out_specs=[_qspec] * NQ,
    out_shape=[_qshape] * NQ,
)


def _tc_post_body(*refs):
    zs = refs[:NQ]
    dinv, conv_b, fc2_W, fc2_b, out = refs[NQ:]
    di = dinv[...]
    z = jnp.concatenate([zq[...] for zq in zs], axis=1)
    h = jnp.maximum(z * di + conv_b[...], 0.0)
    out[...] = jnp.dot(h, fc2_W[...], preferred_element_type=jnp.float32) + fc2_b[...]


_tc_post = pl.pallas_call(
    _tc_post_body,
    grid=(GRID,),
    in_specs=[_qspec] * NQ + [
        _dspec,
        pl.BlockSpec((1, WIDTH), lambda i: (0, 0)),
        pl.BlockSpec((WIDTH, 1), lambda i: (0, 0)),
        pl.BlockSpec((1, 1), lambda i: (0, 0)),
    ],
    out_specs=pl.BlockSpec((BN, 1), lambda i: (i, 0)),
    out_shape=jax.ShapeDtypeStruct((N_PAD, 1), jnp.float32),
)


def kernel(x, edge_index, fc1_W, fc1_b, conv_W, conv_b, fc2_W, fc2_b):
    src = jnp.concatenate(
        [edge_index[0], jnp.zeros((E_PAD - E,), jnp.int32)])
    dst = jnp.concatenate(
        [edge_index[1], jnp.full((E_PAD - E,), N, jnp.int32)]).reshape(EROWS, LANES)
    x_pad = jnp.concatenate([x, jnp.zeros((N_PAD - N, 3), x.dtype)], axis=0)

    sc_degree, sc_scatter = _sc_kernels()
    fc1_b2 = fc1_b.reshape(1, WIDTH)
    conv_b2 = conv_b.reshape(1, WIDTH)

    degp = sc_degree(dst, jnp.ones((N_PAD, DW), jnp.float32))
    outs = _tc_pre(x_pad, degp, fc1_W, fc1_b2, conv_W)
    ys, dinv = outs[:NQ], outs[NQ]
    for layer in range(DEPTH):
        zs = sc_scatter(*ys, src, dst)
        if layer < DEPTH - 1:
            ys = _tc_mid(*zs, dinv, conv_W, conv_b2)
    out = _tc_post(*zs, dinv, conv_b2, fc2_W, fc2_b.reshape(1, 1))
    return out[:N]


# speedup vs baseline: 1.0351x; 1.0022x over previous
import functools

import jax
import jax.numpy as jnp
from jax import lax
from jax.experimental import pallas as pl
from jax.experimental.pallas import tpu as pltpu
from jax.experimental.pallas import tpu_sc as plsc

N = 50000
E = 800000
WIDTH = 64
NQ = 2
QW = WIDTH // NQ
DEPTH = 4

LANES = 128
N_PAD = 50176
EROWS = 6400
E_PAD = EROWS * LANES
RPT = N_PAD // 16
ERPT = EROWS // 16
EPT = ERPT * LANES
SUP = 1024
SPT = EPT // SUP
SROWS = SUP // LANES
GC = 256
GPS = SUP // GC


EDPT = EROWS // 32
DW = 8


def _sc_degree_body(dst2d, ones2d, degp, dstv, onesv, d_sh, sem):
    cid = lax.axis_index("c")
    sid = lax.axis_index("s")
    pltpu.sync_copy(ones2d.at[pl.ds(sid * RPT, RPT)],
                    d_sh.at[pl.ds(sid * RPT, RPT)])
    pltpu.sync_copy(ones2d.at[pl.ds(0, LANES)], onesv)
    pltpu.sync_copy(
        dst2d.at[pl.ds(cid * (EROWS // 2) + sid * EDPT, EDPT)], dstv)
    plsc.subcore_barrier()

    def step(j, carry):
        pltpu.sync_copy(onesv, d_sh.at[dstv.at[j]], add=True)
        return carry

    lax.fori_loop(0, EDPT, step, 0)
    plsc.subcore_barrier()

    def writeout(c):
        pltpu.sync_copy(d_sh.at[pl.ds(sid * RPT, RPT)],
                        degp.at[c, pl.ds(sid * RPT, RPT)])

    pl.when(cid == 0)(lambda: writeout(0))
    pl.when(cid == 1)(lambda: writeout(1))


def _sc_scatter_body(*refs):
    y0, y1, src_flat, dst_flat, z0, z1 = refs[:6]
    srcb = refs[6:8]
    dstb = refs[8:10]
    rows = refs[10:12]
    z_sh = refs[12]
    isem = refs[13:15]
    gsem = refs[15:17]
    ssem = refs[17:19]
    cid = lax.axis_index("c")
    sid = lax.axis_index("s")
    src_base = sid * EPT

    def idxload(s, h):
        pltpu.async_copy(src_flat.at[pl.ds(src_base + s * SUP, SUP)],
                         srcb[h], isem[h])
        pltpu.async_copy(dst_flat.at[pl.ds(src_base + s * SUP, SUP)],
                         dstb[h], isem[h])

    def idxwait(s, h):
        pltpu.make_async_copy(src_flat.at[pl.ds(src_base + s * SUP, SUP)],
                              srcb[h], isem[h]).wait()
        pltpu.make_async_copy(dst_flat.at[pl.ds(src_base + s * SUP, SUP)],
                              dstb[h], isem[h]).wait()

    def half(y_hbm, z_hbm):
        idxload(0, 0)
        idxload(1, 1)
        pltpu.sync_copy(y_hbm.at[pl.ds(sid * RPT, RPT)],
                        z_sh.at[pl.ds(sid * RPT, RPT)])

        def gather(h, q, rb):
            pltpu.async_copy(
                y_hbm.at[srcb[h].at[pl.ds(q * GC, GC)]], rows[rb], gsem[rb])

        def gwait(h, q, rb):
            pltpu.make_async_copy(
                y_hbm.at[srcb[h].at[pl.ds(q * GC, GC)]],
                rows[rb], gsem[rb]).wait()

        def scatter_issue(h, q, rb):
            pltpu.async_copy(
                rows[rb], z_sh.at[dstb[h].at[pl.ds(q * GC, GC)]],
                ssem[rb], add=True)

        def scatter_wait(h, rb):
            pltpu.make_async_copy(
                rows[rb], z_sh.at[dstb[h].at[pl.ds(0, GC)]],
                ssem[rb]).wait()

        idxwait(0, 0)
        plsc.subcore_barrier()
        gather(0, 0, 0)

        def process(s, h, maybe_first):
            for q in range(GPS):
                rb = q % 2
                gwait(h, q, rb)
                scatter_issue(h, q, rb)
                if q == 0 and maybe_first:
                    @pl.when(s > 0)
                    def _():
                        scatter_wait(h, 1 - rb)
                else:
                    scatter_wait(h, 1 - rb)
                if q < GPS - 1:
                    gather(h, q + 1, 1 - rb)
                else:
                    @pl.when(s + 1 < SPT)
                    def _(h=h, rb=rb):
                        idxwait(s + 1, 1 - h)
                        gather(1 - h, 0, 1 - rb)

            @pl.when(s + 2 < SPT)
            def _(s=s, h=h):
                idxload(s + 2, h)

        def superpair(p, carry):
            process(2 * p, 0, True)
            process(2 * p + 1, 1, False)
            return carry

        lax.fori_loop(0, SPT // 2, superpair, 0)
        scatter_wait(1, 1)
        plsc.subcore_barrier()
        pltpu.sync_copy(z_sh.at[pl.ds(sid * RPT, RPT)],
                        z_hbm.at[pl.ds(sid * RPT, RPT)])

    pl.when(cid == 0)(lambda: half(y0, z0))
    pl.when(cid == 1)(lambda: half(y1, z1))


@functools.cache
def _sc_kernels():
    mesh = plsc.VectorSubcoreMesh(core_axis_name="c", subcore_axis_name="s")
    params = pltpu.CompilerParams(use_tc_tiling_on_sc=False)
    degree = pl.kernel(
        _sc_degree_body,
        out_type=jax.ShapeDtypeStruct((2, N_PAD, DW), jnp.float32),
        mesh=mesh,
        scratch_types=[
            pltpu.VMEM((EDPT, LANES), jnp.int32),
            pltpu.VMEM((LANES, DW), jnp.float32),
            pltpu.VMEM_SHARED((N_PAD, DW), jnp.float32),
            pltpu.SemaphoreType.DMA,
        ],
        compiler_params=params,
    )
    qshape = jax.ShapeDtypeStruct((N_PAD, QW), jnp.float32)
    scatter = pl.kernel(
        _sc_scatter_body,
        out_type=[qshape] * NQ,
        mesh=mesh,
        scratch_types=(
            [pltpu.VMEM((SUP,), jnp.int32)] * 2
            + [pltpu.VMEM((SUP,), jnp.int32)] * 2
            + [pltpu.VMEM((GC, QW), jnp.float32)] * 2
            + [pltpu.VMEM_SHARED((N_PAD, QW), jnp.float32)]
            + [pltpu.SemaphoreType.DMA] * 6
        ),
        compiler_params=params,
    )
    return degree, scatter


BN = 3136
GRID = N_PAD // BN


def _split(y, outs):
    for q, ref in enumerate(outs):
        ref[...] = y[:, q * QW:(q + 1) * QW]


def _tc_pre_body(x, degp, fc1_W, fc1_b, conv_W, *outs):
    di = lax.rsqrt(degp[0, :, 0:1] + degp[1, :, 0:1] - 1.0)
    h = jnp.dot(x[...], fc1_W[...], preferred_element_type=jnp.float32)
    h = h + fc1_b[...]
    y = jnp.dot(h, conv_W[...], preferred_element_type=jnp.float32) * di
    _split(y, outs[:NQ])
    outs[NQ][...] = di


_qspec = pl.BlockSpec((BN, QW), lambda i: (i, 0))
_qshape = jax.ShapeDtypeStruct((N_PAD, QW), jnp.float32)
_dspec = pl.BlockSpec((BN, 1), lambda i: (i, 0))

_tc_pre = pl.pallas_call(
    _tc_pre_body,
    grid=(GRID,),
    in_specs=[
        pl.BlockSpec((BN, 3), lambda i: (i, 0)),
        pl.BlockSpec((2, BN, DW), lambda i: (0, i, 0)),
        pl.BlockSpec((3, WIDTH), lambda i: (0, 0)),
        pl.BlockSpec((1, WIDTH), lambda i: (0, 0)),
        pl.BlockSpec((WIDTH, WIDTH), lambda i: (0, 0)),
    ],
    out_specs=[_qspec] * NQ + [_dspec],
    out_shape=[_qshape] * NQ + [jax.ShapeDtypeStruct((N_PAD, 1), jnp.float32)],
)


def _tc_mid_body(*refs):
    zs = refs[:NQ]
    dinv, conv_W, conv_b = refs[NQ:NQ + 3]
    ys = refs[NQ + 3:]
    di = dinv[...]
    z = jnp.concatenate([zq[...] for zq in zs], axis=1)
    h = jnp.maximum(z * di + conv_b[...], 0.0)
    y = jnp.dot(h, conv_W[...], preferred_element_type=jnp.float32) * di
    _split(y, ys)


_tc_mid = pl.pallas_call(
    _tc_mid_body,
    grid=(GRID,),
    in_specs=[_qspec] * NQ + [
        _dspec,
        pl.BlockSpec((WIDTH, WIDTH), lambda i: (0, 0)),
        pl.BlockSpec((1, WIDTH), lambda i: (0, 0)),
    ],
    out_specs=[_qspec] * NQ,
    out_shape=[_qshape] * NQ,
)


def _tc_post_body(*refs):
    zs = refs[:NQ]
    dinv, conv_b, fc2_W, fc2_b, out = refs[NQ:]
    di = dinv[...]
    z = jnp.concatenate([zq[...] for zq in zs], axis=1)
    h = jnp.maximum(z * di + conv_b[...], 0.0)
    out[...] = jnp.dot(h, fc2_W[...], preferred_element_type=jnp.float32) + fc2_b[...]


_tc_post = pl.pallas_call(
    _tc_post_body,
    grid=(GRID,),
    in_specs=[_qspec] * NQ + [
        _dspec,
        pl.BlockSpec((1, WIDTH), lambda i: (0, 0)),
        pl.BlockSpec((WIDTH, 1), lambda i: (0, 0)),
        pl.BlockSpec((1, 1), lambda i: (0, 0)),
    ],
    out_specs=pl.BlockSpec((BN, 1), lambda i: (i, 0)),
    out_shape=jax.ShapeDtypeStruct((N_PAD, 1), jnp.float32),
)


def kernel(x, edge_index, fc1_W, fc1_b, conv_W, conv_b, fc2_W, fc2_b):
    src = jnp.concatenate(
        [edge_index[0], jnp.zeros((E_PAD - E,), jnp.int32)])
    dst = jnp.concatenate(
        [edge_index[1], jnp.full((E_PAD - E,), N, jnp.int32)])
    dst2d = dst.reshape(EROWS, LANES)
    x_pad = jnp.concatenate([x, jnp.zeros((N_PAD - N, 3), x.dtype)], axis=0)

    sc_degree, sc_scatter = _sc_kernels()
    fc1_b2 = fc1_b.reshape(1, WIDTH)
    conv_b2 = conv_b.reshape(1, WIDTH)

    degp = sc_degree(dst2d, jnp.ones((N_PAD, DW), jnp.float32))
    outs = _tc_pre(x_pad, degp, fc1_W, fc1_b2, conv_W)
    ys, dinv = outs[:NQ], outs[NQ]
    for layer in range(DEPTH):
        zs = sc_scatter(*ys, src, dst)
        if layer < DEPTH - 1:
            ys = _tc_mid(*zs, dinv, conv_W, conv_b2)
    out = _tc_post(*zs, dinv, conv_b2, fc2_W, fc2_b.reshape(1, 1))
    return out[:N]


# speedup vs baseline: 1.0360x; 1.0009x over previous
import functools

import jax
import jax.numpy as jnp
from jax import lax
from jax.experimental import pallas as pl
from jax.experimental.pallas import tpu as pltpu
from jax.experimental.pallas import tpu_sc as plsc

N = 50000
E = 800000
WIDTH = 64
NQ = 2
QW = WIDTH // NQ
DEPTH = 4

LANES = 128
N_PAD = 50176
EROWS = 6400
E_PAD = EROWS * LANES
RPT = N_PAD // 16
ERPT = EROWS // 16
EPT = ERPT * LANES
SUP = 1024
SPT = EPT // SUP
GC = 256
GPS = SUP // GC


EDPT = EROWS // 32
DW = 8


def _sc_degree_body(dst2d, ones2d, degp, dstv, onesv, d_sh, sem):
    cid = lax.axis_index("c")
    sid = lax.axis_index("s")
    pltpu.sync_copy(ones2d.at[pl.ds(sid * RPT, RPT)],
                    d_sh.at[pl.ds(sid * RPT, RPT)])
    pltpu.sync_copy(ones2d.at[pl.ds(0, LANES)], onesv)
    pltpu.sync_copy(
        dst2d.at[pl.ds(cid * (EROWS // 2) + sid * EDPT, EDPT)], dstv)
    plsc.subcore_barrier()

    def step(j, carry):
        pltpu.sync_copy(onesv, d_sh.at[dstv.at[j]], add=True)
        return carry

    lax.fori_loop(0, EDPT, step, 0)
    plsc.subcore_barrier()

    def writeout(c):
        pltpu.sync_copy(d_sh.at[pl.ds(sid * RPT, RPT)],
                        degp.at[c, pl.ds(sid * RPT, RPT)])

    pl.when(cid == 0)(lambda: writeout(0))
    pl.when(cid == 1)(lambda: writeout(1))


def _sc_scatter_body(*refs):
    y0, y1, src_flat, dst_flat, z0, z1 = refs[:6]
    srcb = refs[6:8]
    dstb = refs[8:10]
    rows = refs[10:12]
    z_sh = refs[12]
    isem = refs[13:15]
    gsem = refs[15:17]
    ssem = refs[17:19]
    cid = lax.axis_index("c")
    sid = lax.axis_index("s")
    src_base = sid * EPT

    def idxload(s, h):
        pltpu.async_copy(src_flat.at[pl.ds(src_base + s * SUP, SUP)],
                         srcb[h], isem[h])
        pltpu.async_copy(dst_flat.at[pl.ds(src_base + s * SUP, SUP)],
                         dstb[h], isem[h])

    def idxwait(s, h):
        pltpu.make_async_copy(src_flat.at[pl.ds(src_base + s * SUP, SUP)],
                              srcb[h], isem[h]).wait()
        pltpu.make_async_copy(dst_flat.at[pl.ds(src_base + s * SUP, SUP)],
                              dstb[h], isem[h]).wait()

    def half(y_hbm, z_hbm):
        idxload(0, 0)
        idxload(1, 1)
        pltpu.sync_copy(y_hbm.at[pl.ds(sid * RPT, RPT)],
                        z_sh.at[pl.ds(sid * RPT, RPT)])

        def gather(h, q, rb):
            pltpu.async_copy(
                y_hbm.at[srcb[h].at[pl.ds(q * GC, GC)]], rows[rb], gsem[rb])

        def gwait(h, q, rb):
            pltpu.make_async_copy(
                y_hbm.at[srcb[h].at[pl.ds(q * GC, GC)]],
                rows[rb], gsem[rb]).wait()

        def scatter_issue(h, q, rb):
            pltpu.async_copy(
                rows[rb], z_sh.at[dstb[h].at[pl.ds(q * GC, GC)]],
                ssem[rb], add=True)

        def scatter_wait(h, rb):
            pltpu.make_async_copy(
                rows[rb], z_sh.at[dstb[h].at[pl.ds(0, GC)]],
                ssem[rb]).wait()

        idxwait(0, 0)
        plsc.subcore_barrier()
        gather(0, 0, 0)

        def process(s, h, maybe_first):
            for q in range(GPS):
                rb = q % 2
                gwait(h, q, rb)
                scatter_issue(h, q, rb)
                if q == 0 and maybe_first:
                    @pl.when(s > 0)
                    def _():
                        scatter_wait(h, 1 - rb)
                else:
                    scatter_wait(h, 1 - rb)
                if q < GPS - 1:
                    gather(h, q + 1, 1 - rb)
                else:
                    @pl.when(s + 1 < SPT)
                    def _(h=h, rb=rb):
                        idxwait(s + 1, 1 - h)
                        gather(1 - h, 0, 1 - rb)

            @pl.when(s + 2 < SPT)
            def _(s=s, h=h):
                idxload(s + 2, h)

        def superpair(p, carry):
            process(2 * p, 0, True)
            process(2 * p + 1, 1, False)
            return carry

        lax.fori_loop(0, SPT // 2, superpair, 0)
        scatter_wait(1, 1)
        plsc.subcore_barrier()
        pltpu.sync_copy(z_sh.at[pl.ds(sid * RPT, RPT)],
                        z_hbm.at[pl.ds(sid * RPT, RPT)])

    pl.when(cid == 0)(lambda: half(y0, z0))
    pl.when(cid == 1)(lambda: half(y1, z1))


@functools.cache
def _sc_kernels():
    mesh = plsc.VectorSubcoreMesh(core_axis_name="c", subcore_axis_name="s")
    params = pltpu.CompilerParams(use_tc_tiling_on_sc=False)
    degree = pl.kernel(
        _sc_degree_body,
        out_type=jax.ShapeDtypeStruct((2, N_PAD, DW), jnp.float32),
        mesh=mesh,
        scratch_types=[
            pltpu.VMEM((EDPT, LANES), jnp.int32),
            pltpu.VMEM((LANES, DW), jnp.float32),
            pltpu.VMEM_SHARED((N_PAD, DW), jnp.float32),
            pltpu.SemaphoreType.DMA,
        ],
        compiler_params=params,
    )
    qshape = jax.ShapeDtypeStruct((N_PAD, QW), jnp.float32)
    scatter = pl.kernel(
        _sc_scatter_body,
        out_type=[qshape] * NQ,
        mesh=mesh,
        scratch_types=(
            [pltpu.VMEM((SUP,), jnp.int32)] * 2
            + [pltpu.VMEM((SUP,), jnp.int32)] * 2
            + [pltpu.VMEM((GC, QW), jnp.float32)] * 2
            + [pltpu.VMEM_SHARED((N_PAD, QW), jnp.float32)]
            + [pltpu.SemaphoreType.DMA] * 6
        ),
        compiler_params=params,
    )
    return degree, scatter


BN = 3136
GRID = N_PAD // BN


def _split(y, outs):
    for q, ref in enumerate(outs):
        ref[...] = y[:, q * QW:(q + 1) * QW]


def _tc_pre_body(x, degp, fc1_W, fc1_b, conv_W, *outs):
    di = lax.rsqrt(degp[0, :, 0:1] + degp[1, :, 0:1] - 1.0)
    h = jnp.dot(x[...], fc1_W[...], preferred_element_type=jnp.float32)
    h = h + fc1_b[...]
    y = jnp.dot(h, conv_W[...], preferred_element_type=jnp.float32) * di
    _split(y, outs[:NQ])
    outs[NQ][...] = di


_qspec = pl.BlockSpec((BN, QW), lambda i: (i, 0))
_qshape = jax.ShapeDtypeStruct((N_PAD, QW), jnp.float32)
_dspec = pl.BlockSpec((BN, 1), lambda i: (i, 0))

_tc_pre = pl.pallas_call(
    _tc_pre_body,
    grid=(GRID,),
    in_specs=[
        pl.BlockSpec((BN, 3), lambda i: (i, 0)),
        pl.BlockSpec((2, BN, DW), lambda i: (0, i, 0)),
        pl.BlockSpec((3, WIDTH), lambda i: (0, 0)),
        pl.BlockSpec((1, WIDTH), lambda i: (0, 0)),
        pl.BlockSpec((WIDTH, WIDTH), lambda i: (0, 0)),
    ],
    out_specs=[_qspec] * NQ + [_dspec],
    out_shape=[_qshape] * NQ + [jax.ShapeDtypeStruct((N_PAD, 1), jnp.float32)],
)


def _tc_mid_body(*refs):
    zs = refs[:NQ]
    dinv, conv_W, conv_b = refs[NQ:NQ + 3]
    ys = refs[NQ + 3:]
    di = dinv[...]
    z = jnp.concatenate([zq[...] for zq in zs], axis=1)
    h = jnp.maximum(z * di + conv_b[...], 0.0)
    y = jnp.dot(h, conv_W[...], preferred_element_type=jnp.float32) * di
    _split(y, ys)


_tc_mid = pl.pallas_call(
    _tc_mid_body,
    grid=(GRID,),
    in_specs=[_qspec] * NQ + [
        _dspec,
        pl.BlockSpec((WIDTH, WIDTH), lambda i: (0, 0)),
        pl.BlockSpec((1, WIDTH), lambda i: (0, 0)),
    ],
    out_specs=[_qspec] * NQ,
    out_shape=[_qshape] * NQ,
)


def _tc_post_body(*refs):
    zs = refs[:NQ]
    dinv, conv_b, fc2_W, fc2_b, out = refs[NQ:]
    di = dinv[...]
    z = jnp.concatenate([zq[...] for zq in zs], axis=1)
    h = jnp.maximum(z * di + conv_b[...], 0.0)
    out[...] = jnp.dot(h, fc2_W[...], preferred_element_type=jnp.float32) + fc2_b[...]


_tc_post = pl.pallas_call(
    _tc_post_body,
    grid=(GRID,),
    in_specs=[_qspec] * NQ + [
        _dspec,
        pl.BlockSpec((1, WIDTH), lambda i: (0, 0)),
        pl.BlockSpec((WIDTH, 1), lambda i: (0, 0)),
        pl.BlockSpec((1, 1), lambda i: (0, 0)),
    ],
    out_specs=pl.BlockSpec((BN, 1), lambda i: (i, 0)),
    out_shape=jax.ShapeDtypeStruct((N_PAD, 1), jnp.float32),
)


def kernel(x, edge_index, fc1_W, fc1_b, conv_W, conv_b, fc2_W, fc2_b):
    src = jnp.concatenate(
        [edge_index[0], jnp.zeros((E_PAD - E,), jnp.int32)])
    dst = jnp.concatenate(
        [edge_index[1], jnp.full((E_PAD - E,), N, jnp.int32)])
    dst2d = dst.reshape(EROWS, LANES)
    x_pad = jnp.concatenate([x, jnp.zeros((N_PAD - N, 3), x.dtype)], axis=0)

    sc_degree, sc_scatter = _sc_kernels()
    fc1_b2 = fc1_b.reshape(1, WIDTH)
    conv_b2 = conv_b.reshape(1, WIDTH)

    degp = sc_degree(dst2d, jnp.ones((N_PAD, DW), jnp.float32))
    outs = _tc_pre(x_pad, degp, fc1_W, fc1_b2, conv_W)
    ys, dinv = outs[:NQ], outs[NQ]
    for layer in range(DEPTH):
        zs = sc_scatter(*ys, src, dst)
        if layer < DEPTH - 1:
            ys = _tc_mid(*zs, dinv, conv_W, conv_b2)
    out = _tc_post(*zs, dinv, conv_b2, fc2_W, fc2_b.reshape(1, 1))
    return out[:N]
